# Initial kernel scaffold; baseline (speedup 1.0000x reference)
import jax
import jax.numpy as jnp
from jax.experimental import pallas as pl


def kernel(x, conv1_w, conv1_b, bn1_g, bn1_b, conv2_w, conv2_b, bn2_g, bn2_b, conv3_w, conv3_b, bn3_g, bn3_b, fc_w, fc_b, g1_w, g1_b, ln1_g, ln1_b, g2_w, g2_b, ln2_g, ln2_b, g3_w, g3_b):
    raise NotImplementedError("write your pallas kernel here")


# speedup vs baseline: 1.2806x; 1.2806x over previous
import jax
import jax.numpy as jnp
from jax.experimental import pallas as pl

_G = 8
_TEMP = 0.5
_EPS = 1e-5


def _dot_kernel(a_ref, w_ref, b_ref, o_ref):
    o_ref[...] = jnp.dot(a_ref[...], w_ref[...],
                         preferred_element_type=jnp.float32) + b_ref[...]


def _dot_bias(a, w, b):
    n, _ = a.shape
    return pl.pallas_call(
        _dot_kernel,
        out_shape=jax.ShapeDtypeStruct((n, w.shape[1]), jnp.float32),
    )(a, w, b)


def _route_kernel(assign_ref, x_ref, out_ref):
    a = assign_ref[0]
    xb = x_ref[0]
    p = a.shape[0]
    ppg = p // _G
    eye8 = (jax.lax.broadcasted_iota(jnp.int32, (_G, 128), 0)
            == jax.lax.broadcasted_iota(jnp.int32, (_G, 128), 1))
    gw_rows = jax.lax.dot_general(eye8.astype(jnp.float32), a,
                                  (((1,), (1,)), ((), ())),
                                  precision=jax.lax.Precision.HIGHEST,
                                  preferred_element_type=jnp.float32)
    ii = jax.lax.broadcasted_iota(jnp.int32, (p, p), 0)
    jj = jax.lax.broadcasted_iota(jnp.int32, (p, p), 1)
    kk = jax.lax.broadcasted_iota(jnp.int32, (ppg, p), 0)
    blocks = []
    for g in range(_G):
        col_g = a[:, g:g + 1]
        row_g = gw_rows[g:g + 1, :]
        beats = (col_g > row_g) | ((col_g == row_g) & (ii < jj))
        rank = jnp.sum(beats.astype(jnp.int32), axis=0, keepdims=True)
        blocks.append((kk == rank).astype(jnp.float32) * row_g)
    sel = jnp.concatenate(blocks, axis=0)
    out_ref[0] = jnp.dot(sel, xb, precision=jax.lax.Precision.HIGHEST,
                         preferred_element_type=jnp.float32)


def _route(assign_p, xflat):
    b, p, hw = xflat.shape
    return pl.pallas_call(
        _route_kernel,
        grid=(b,),
        in_specs=[
            pl.BlockSpec((1, p, 128), lambda i: (i, 0, 0)),
            pl.BlockSpec((1, p, hw), lambda i: (i, 0, 0)),
        ],
        out_specs=pl.BlockSpec((1, p, hw), lambda i: (i, 0, 0)),
        out_shape=jax.ShapeDtypeStruct((b, p, hw), jnp.float32),
    )(assign_p, xflat)


def kernel(x, conv1_w, conv1_b, bn1_g, bn1_b, conv2_w, conv2_b, bn2_g, bn2_b,
           conv3_w, conv3_b, bn3_g, bn3_b, fc_w, fc_b,
           g1_w, g1_b, ln1_g, ln1_b, g2_w, g2_b, ln2_g, ln2_b, g3_w, g3_b):
    B, P, C, H, W = x.shape
    n_img = B * P

    h = x.reshape(n_img, C, H, W)
    for cw, cb, bg, bb in ((conv1_w, conv1_b, bn1_g, bn1_b),
                           (conv2_w, conv2_b, bn2_g, bn2_b),
                           (conv3_w, conv3_b, bn3_g, bn3_b)):
        y = jax.lax.conv_general_dilated(
            h, cw, window_strides=(1, 1), padding='SAME',
            dimension_numbers=('NCHW', 'OIHW', 'NCHW')) + cb[None, :, None, None]
        m = jnp.mean(y, axis=(0, 2, 3), keepdims=True)
        v = jnp.var(y, axis=(0, 2, 3), keepdims=True)
        h = jax.nn.relu((y - m) / jnp.sqrt(v + _EPS)
                        * bg[None, :, None, None] + bb[None, :, None, None])
    pooled = jnp.mean(h, axis=(2, 3))

    feat = _dot_bias(pooled, fc_w.T, fc_b.reshape(1, -1))

    def _ln(z, g, b):
        mm = jnp.mean(z, axis=-1, keepdims=True)
        vv = jnp.var(z, axis=-1, keepdims=True)
        return (z - mm) / jnp.sqrt(vv + _EPS) * g + b

    z = jax.nn.relu(_ln(_dot_bias(feat, g1_w.T, g1_b.reshape(1, -1)),
                        ln1_g, ln1_b))
    z = jax.nn.relu(_ln(_dot_bias(z, g2_w.T, g2_b.reshape(1, -1)),
                        ln2_g, ln2_b))
    g3p = jnp.zeros((g3_w.shape[1], 128), jnp.float32).at[:, :_G].set(g3_w.T)
    b3p = jnp.zeros((1, 128), jnp.float32).at[:, :_G].set(g3_b)
    logits = _dot_bias(z, g3p, b3p)[:, :_G] / _TEMP
    assignment = jax.nn.softmax(logits, axis=-1).reshape(B, P, _G)

    assign_p = jnp.zeros((B, P, 128), jnp.float32).at[:, :, :_G].set(assignment)
    grouped = _route(assign_p, x.reshape(B, P, H * W))
    output = grouped.reshape(B, _G, P // _G, C, H, W)
    return output, assignment
